# baseline (speedup 1.0000x reference)
import functools

import jax
import jax.numpy as jnp
from jax import lax
from jax.experimental import pallas as pl
from jax.experimental.pallas import tpu as pltpu
from jax.experimental.pallas import tpu_sc as plsc

_N = 10000
_E = 160000
_D = 256
_LANES = 16
_VPR = _D // _LANES

_NW = 32
_NPT = 313
_NPAD = _NW * _NPT

_EPWP = 5120
_EPAD = _NW * _EPWP
_GCH = 32
_GNC = _EPWP // _GCH
_GR = 5
_GAHEAD = _GR - 2

_SCH = 80
_EQPAD = _E + 128

_sc_info = plsc.get_sparse_core_info()
_mesh = plsc.VectorSubcoreMesh(core_axis_name="c", subcore_axis_name="s")


def _node_mm_body(h_ref, posp_ref, wah_ref, wap_ref, ba_ref, a_ref, c_ref):
    cb = jnp.dot(posp_ref[...], wap_ref[...], preferred_element_type=jnp.float32)
    a_ref[...] = (
        jnp.dot(h_ref[...], wah_ref[...], preferred_element_type=jnp.float32)
        + cb + ba_ref[...]
    )
    c_ref[...] = cb


def _node_mm(h, posp, wah, wap, ba):
    br = 400
    return pl.pallas_call(
        _node_mm_body,
        grid=(_N // br,),
        in_specs=[
            pl.BlockSpec((br, _D), lambda i: (i, 0)),
            pl.BlockSpec((br, 128), lambda i: (i, 0)),
            pl.BlockSpec((_D, _D), lambda i: (0, 0)),
            pl.BlockSpec((128, _D), lambda i: (0, 0)),
            pl.BlockSpec((1, _D), lambda i: (0, 0)),
        ],
        out_specs=[
            pl.BlockSpec((br, _D), lambda i: (i, 0)),
            pl.BlockSpec((br, _D), lambda i: (i, 0)),
        ],
        out_shape=[
            jax.ShapeDtypeStruct((_N, _D), jnp.float32),
            jax.ShapeDtypeStruct((_N, _D), jnp.float32),
        ],
    )(h, posp, wah, wap, ba)


def _edge_mm_body(ga_ref, gc_ref, wb_ref, q_ref):
    g = jnp.maximum(ga_ref[...] - gc_ref[...], 0.0)
    q_ref[...] = jnp.dot(g, wb_ref[...], preferred_element_type=jnp.float32)


def _edge_mm(ga, gc, wb):
    br = 2000
    return pl.pallas_call(
        _edge_mm_body,
        grid=(_E // br,),
        in_specs=[
            pl.BlockSpec((br, _D), lambda i: (i, 0)),
            pl.BlockSpec((br, _D), lambda i: (i, 0)),
            pl.BlockSpec((_D, _D), lambda i: (0, 0)),
        ],
        out_specs=pl.BlockSpec((br, _D), lambda i: (i, 0)),
        out_shape=jax.ShapeDtypeStruct((_EQPAD, _D), jnp.float32),
    )(ga, gc, wb)


def _sc_gather(a, c, src_p, dst_p):
    @functools.partial(
        pl.kernel,
        mesh=_mesh,
        out_type=[
            jax.ShapeDtypeStruct((_EPAD, _D), jnp.float32),
            jax.ShapeDtypeStruct((_EPAD, _D), jnp.float32),
        ],
        scratch_types=(
            [pltpu.VMEM((_EPWP,), jnp.int32)] * 2
            + [pltpu.VMEM((_GCH, _D), jnp.float32)] * (2 * _GR)
            + [pltpu.SemaphoreType.DMA] * (4 * _GR)
        ),
    )
    def k(a_hbm, c_hbm, src_hbm, dst_hbm, ga_hbm, gc_hbm, sidx, didx, *rest):
        gab = rest[0:_GR]
        gcb = rest[_GR:2 * _GR]
        asem = rest[2 * _GR:3 * _GR]
        csem = rest[3 * _GR:4 * _GR]
        oasem = rest[4 * _GR:5 * _GR]
        ocsem = rest[5 * _GR:6 * _GR]
        wid = lax.axis_index("s") * _sc_info.num_cores + lax.axis_index("c")
        ebase = pl.multiple_of(wid * _EPWP, 8)
        pltpu.sync_copy(src_hbm.at[pl.ds(ebase, _EPWP)], sidx)
        pltpu.sync_copy(dst_hbm.at[pl.ds(ebase, _EPWP)], didx)

        def issue(ci, b):
            off = pl.multiple_of(ci * _GCH, 8)
            pltpu.async_copy(a_hbm.at[sidx.at[pl.ds(off, _GCH)]], gab[b], asem[b])
            pltpu.async_copy(c_hbm.at[didx.at[pl.ds(off, _GCH)]], gcb[b], csem[b])

        for p in range(_GAHEAD):
            issue(p, p)

        def outer(oo, _):
            for sb in range(_GR):
                ci = _GR * oo + sb
                b = sb
                b2 = (sb + _GAHEAD) % _GR
                pltpu.make_async_copy(
                    a_hbm.at[sidx.at[pl.ds(0, _GCH)]], gab[b], asem[b]).wait()
                pltpu.make_async_copy(
                    c_hbm.at[didx.at[pl.ds(0, _GCH)]], gcb[b], csem[b]).wait()
                row = pl.multiple_of(ebase + ci * _GCH, 8)
                pltpu.async_copy(gab[b], ga_hbm.at[pl.ds(row, _GCH)], oasem[b])
                pltpu.async_copy(gcb[b], gc_hbm.at[pl.ds(row, _GCH)], ocsem[b])

                @pl.when(ci >= 2)
                def _():
                    pltpu.make_async_copy(
                        gab[b2], ga_hbm.at[pl.ds(0, _GCH)], oasem[b2]).wait()
                    pltpu.make_async_copy(
                        gcb[b2], gc_hbm.at[pl.ds(0, _GCH)], ocsem[b2]).wait()

                @pl.when(ci + _GAHEAD < _GNC)
                def _():
                    issue(ci + _GAHEAD, b2)
            return 0

        lax.fori_loop(0, _GNC // _GR, outer, 0)
        for off in range(2):
            b = (_GNC - 2 + off) % _GR
            pltpu.make_async_copy(gab[b], ga_hbm.at[pl.ds(0, _GCH)], oasem[b]).wait()
            pltpu.make_async_copy(gcb[b], gc_hbm.at[pl.ds(0, _GCH)], ocsem[b]).wait()

    return k(a, c, src_p, dst_p)


def _sc_scatter(q, dstp, eb, bb):
    @functools.partial(
        pl.kernel,
        mesh=_mesh,
        out_type=jax.ShapeDtypeStruct((_NPAD * _D,), jnp.float32),
        scratch_types=[
            pltpu.VMEM((_NPT * _D,), jnp.float32),
            pltpu.VMEM((_SCH, _D), jnp.float32),
            pltpu.VMEM((_SCH, _D), jnp.float32),
            pltpu.VMEM((_SCH + 32,), jnp.int32),
            pltpu.VMEM((_SCH + 32,), jnp.int32),
            pltpu.VMEM((16,), jnp.int32),
            pltpu.VMEM((_D,), jnp.float32),
            pltpu.SemaphoreType.DMA,
            pltpu.SemaphoreType.DMA,
            pltpu.SemaphoreType.DMA,
            pltpu.SemaphoreType.DMA,
        ],
    )
    def k(q_hbm, dst_hbm, eb_hbm, bb_hbm, out_hbm,
          tbl, qb0, qb1, db0, db1, ebuf, bbuf,
          qsem0, qsem1, dsem0, dsem1):
        qb = (qb0, qb1)
        db = (db0, db1)
        qsem = (qsem0, qsem1)
        dsem = (dsem0, dsem1)
        wid = lax.axis_index("s") * _sc_info.num_cores + lax.axis_index("c")
        nlo = wid * _NPT
        pltpu.sync_copy(eb_hbm.at[pl.ds(pl.multiple_of(wid * 16, 8), 16)], ebuf)
        pltpu.sync_copy(bb_hbm, bbuf)
        ev = ebuf[pl.ds(0, _LANES)]
        elo = ev[0]
        ehi = ev[1]

        neg = jnp.full((_LANES,), -jnp.inf, jnp.float32)

        def initrow(r, _):
            tbl[pl.ds(r * _LANES, _LANES)] = neg
            return 0

        lax.fori_loop(0, _NPT * _VPR, initrow, 0)

        base8 = pl.multiple_of((elo // 8) * 8, 8)
        nchunks = (ehi - base8 + _SCH - 1) // _SCH

        def issue(ci, b):
            s = pl.multiple_of(base8 + ci * _SCH, 8)
            pltpu.async_copy(q_hbm.at[pl.ds(s, _SCH)], qb[b], qsem[b])
            pltpu.async_copy(
                dst_hbm.at[pl.ds(s, _SCH)], db[b].at[pl.ds(0, _SCH)], dsem[b])

        @pl.when(nchunks > 0)
        def _():
            issue(0, 0)

        @pl.when(nchunks > 1)
        def _():
            issue(1, 1)

        def outer(oo, _):
            for b in range(2):
                ci = 2 * oo + b

                @pl.when(ci < nchunks)
                def _():
                    pltpu.make_async_copy(
                        q_hbm.at[pl.ds(0, _SCH)], qb[b], qsem[b]).wait()
                    pltpu.make_async_copy(
                        dst_hbm.at[pl.ds(0, _SCH)],
                        db[b].at[pl.ds(0, _SCH)], dsem[b]).wait()
                    start = pl.multiple_of(base8 + ci * _SCH, 8)
                    i_lo = jnp.maximum(elo - start, 0)
                    i_hi = jnp.minimum(ehi - start, _SCH)

                    def flush(row, acc):
                        base = row * _D
                        for j in range(_VPR):
                            sl = pl.ds(base + j * _LANES, _LANES)
                            tbl[sl] = jnp.maximum(tbl[sl], acc[j])

                    zero = jnp.zeros((_LANES,), jnp.float32)

                    def group(gi, carry):
                        prev = carry[0]
                        acc = carry[1]
                        i0 = gi * _LANES
                        dvec = db[b][pl.ds(i0, _LANES)]
                        for lane in range(_LANES):
                            i = i0 + lane
                            d = dvec[lane]
                            valid = (i >= i_lo) & (i < i_hi)
                            is_new = valid & (d != prev)

                            @pl.when(is_new & (prev >= 0))
                            def _():
                                flush(prev - nlo, acc)

                            qrow = tuple(
                                qb[b][i, pl.ds(j * _LANES, _LANES)]
                                for j in range(_VPR))
                            acc = tuple(
                                jnp.where(
                                    is_new, qrow[j],
                                    jnp.where(valid,
                                              jnp.maximum(acc[j], qrow[j]),
                                              acc[j]))
                                for j in range(_VPR))
                            prev = jnp.where(is_new, d, prev)
                        return (prev, acc)

                    prev, acc = lax.fori_loop(
                        0, _SCH // _LANES, group,
                        (jnp.int32(-1), (zero,) * _VPR))

                    @pl.when(prev >= 0)
                    def _():
                        flush(prev - nlo, acc)

                    @pl.when(ci + 2 < nchunks)
                    def _():
                        issue(ci + 2, b)
            return 0

        lax.fori_loop(0, (nchunks + 1) // 2, outer, 0)

        def finrow(r, _):
            base = r * _D
            for j in range(_VPR):
                sl = pl.ds(base + j * _LANES, _LANES)
                bsl = pl.ds(j * _LANES, _LANES)
                tbl[sl] = jnp.maximum(tbl[sl] + bbuf[bsl], 0.0)
            return 0

        lax.fori_loop(0, _NPT, finrow, 0)
        pltpu.sync_copy(tbl, out_hbm.at[pl.ds(pl.multiple_of(nlo * _D, 8), _NPT * _D)])

    return k(q, dstp, eb, bb)


def kernel(x, pos, edge_index, W1a, b1a, W1b, b1b, W2a, b2a, W2b, b2b):
    src = edge_index[0]
    dst = edge_index[1]
    perm = jnp.argsort(dst)
    dst_s = dst[perm].astype(jnp.int32)
    src_s = src[perm].astype(jnp.int32)
    bnd = jnp.searchsorted(
        dst_s, jnp.arange(_NW + 1, dtype=jnp.int32) * _NPT).astype(jnp.int32)
    eb = jnp.zeros((_NW, 16), jnp.int32)
    eb = eb.at[:, 0].set(bnd[:-1]).at[:, 1].set(bnd[1:]).reshape(-1)
    dstp = jnp.pad(dst_s, (0, _EPAD - _E))
    srcp = jnp.pad(src_s, (0, _EPAD - _E))
    posp = jnp.pad(pos, ((0, 0), (0, 125)))

    h = x
    for (Wa, ba, Wb, bb) in ((W1a, b1a, W1b, b1b), (W2a, b2a, W2b, b2b)):
        wah = Wa[:_D]
        wap = jnp.pad(Wa[_D:], ((0, 125), (0, 0)))
        a, cpart = _node_mm(h, posp, wah, wap, ba[None])
        ga, gc = _sc_gather(a, cpart, srcp, dstp)
        q = _edge_mm(ga, gc, Wb)
        h = _sc_scatter(q, dstp, eb, bb).reshape(_NPAD, _D)[:_N]
    return h


# speedup vs baseline: 1.0326x; 1.0326x over previous
import functools

import jax
import jax.numpy as jnp
from jax import lax
from jax.experimental import pallas as pl
from jax.experimental.pallas import tpu as pltpu
from jax.experimental.pallas import tpu_sc as plsc

_N = 10000
_E = 160000
_D = 256
_LANES = 16
_VPR = _D // _LANES

_NW = 32
_NPT = 313
_NPAD = _NW * _NPT

_EPWP = 5120
_EPAD = _NW * _EPWP
_GCH = 32
_GNC = _EPWP // _GCH
_GR = 5
_GAHEAD = _GR - 2

_SCH = 80
_EQPAD = _E + 128

_sc_info = plsc.get_sparse_core_info()
_mesh = plsc.VectorSubcoreMesh(core_axis_name="c", subcore_axis_name="s")


def _node_mm_body(h_ref, posp_ref, wah_ref, wap_ref, ba_ref, a_ref, c_ref):
    cb = jnp.dot(posp_ref[...], wap_ref[...], preferred_element_type=jnp.float32)
    a_ref[...] = (
        jnp.dot(h_ref[...], wah_ref[...], preferred_element_type=jnp.float32)
        + cb + ba_ref[...]
    )
    c_ref[...] = cb


def _node_mm(h, posp, wah, wap, ba):
    br = 400
    return pl.pallas_call(
        _node_mm_body,
        grid=(_N // br,),
        in_specs=[
            pl.BlockSpec((br, _D), lambda i: (i, 0)),
            pl.BlockSpec((br, 128), lambda i: (i, 0)),
            pl.BlockSpec((_D, _D), lambda i: (0, 0)),
            pl.BlockSpec((128, _D), lambda i: (0, 0)),
            pl.BlockSpec((1, _D), lambda i: (0, 0)),
        ],
        out_specs=[
            pl.BlockSpec((br, _D), lambda i: (i, 0)),
            pl.BlockSpec((br, _D), lambda i: (i, 0)),
        ],
        out_shape=[
            jax.ShapeDtypeStruct((_N, _D), jnp.float32),
            jax.ShapeDtypeStruct((_N, _D), jnp.float32),
        ],
    )(h, posp, wah, wap, ba)


def _edge_mm_body(ga_ref, gc_ref, wb_ref, q_ref):
    g = jnp.maximum(ga_ref[...] - gc_ref[...], 0.0)
    q_ref[...] = jnp.dot(g, wb_ref[...], preferred_element_type=jnp.float32)


def _edge_mm(ga, gc, wb):
    br = 2000
    return pl.pallas_call(
        _edge_mm_body,
        grid=(_E // br,),
        in_specs=[
            pl.BlockSpec((br, _D), lambda i: (i, 0)),
            pl.BlockSpec((br, _D), lambda i: (i, 0)),
            pl.BlockSpec((_D, _D), lambda i: (0, 0)),
        ],
        out_specs=pl.BlockSpec((br, _D), lambda i: (i, 0)),
        out_shape=jax.ShapeDtypeStruct((_EQPAD, _D), jnp.float32),
    )(ga, gc, wb)


def _sc_gather(a, c, src_p, dst_p):
    @functools.partial(
        pl.kernel,
        mesh=_mesh,
        out_type=[
            jax.ShapeDtypeStruct((_EPAD, _D), jnp.float32),
            jax.ShapeDtypeStruct((_EPAD, _D), jnp.float32),
        ],
        scratch_types=(
            [pltpu.VMEM((_EPWP,), jnp.int32)] * 2
            + [pltpu.VMEM((_GCH, _D), jnp.float32)] * (2 * _GR)
            + [pltpu.SemaphoreType.DMA] * (4 * _GR)
        ),
    )
    def k(a_hbm, c_hbm, src_hbm, dst_hbm, ga_hbm, gc_hbm, sidx, didx, *rest):
        gab = rest[0:_GR]
        gcb = rest[_GR:2 * _GR]
        asem = rest[2 * _GR:3 * _GR]
        csem = rest[3 * _GR:4 * _GR]
        oasem = rest[4 * _GR:5 * _GR]
        ocsem = rest[5 * _GR:6 * _GR]
        wid = lax.axis_index("s") * _sc_info.num_cores + lax.axis_index("c")
        ebase = pl.multiple_of(wid * _EPWP, 8)
        pltpu.sync_copy(src_hbm.at[pl.ds(ebase, _EPWP)], sidx)
        pltpu.sync_copy(dst_hbm.at[pl.ds(ebase, _EPWP)], didx)

        def issue(ci, b):
            off = pl.multiple_of(ci * _GCH, 8)
            pltpu.async_copy(a_hbm.at[sidx.at[pl.ds(off, _GCH)]], gab[b], asem[b])
            pltpu.async_copy(c_hbm.at[didx.at[pl.ds(off, _GCH)]], gcb[b], csem[b])

        for p in range(_GAHEAD):
            issue(p, p)

        def outer(oo, _):
            for sb in range(_GR):
                ci = _GR * oo + sb
                b = sb
                b2 = (sb + _GAHEAD) % _GR
                pltpu.make_async_copy(
                    a_hbm.at[sidx.at[pl.ds(0, _GCH)]], gab[b], asem[b]).wait()
                pltpu.make_async_copy(
                    c_hbm.at[didx.at[pl.ds(0, _GCH)]], gcb[b], csem[b]).wait()
                row = pl.multiple_of(ebase + ci * _GCH, 8)
                pltpu.async_copy(gab[b], ga_hbm.at[pl.ds(row, _GCH)], oasem[b])
                pltpu.async_copy(gcb[b], gc_hbm.at[pl.ds(row, _GCH)], ocsem[b])

                @pl.when(ci >= 2)
                def _():
                    pltpu.make_async_copy(
                        gab[b2], ga_hbm.at[pl.ds(0, _GCH)], oasem[b2]).wait()
                    pltpu.make_async_copy(
                        gcb[b2], gc_hbm.at[pl.ds(0, _GCH)], ocsem[b2]).wait()

                @pl.when(ci + _GAHEAD < _GNC)
                def _():
                    issue(ci + _GAHEAD, b2)
            return 0

        lax.fori_loop(0, _GNC // _GR, outer, 0)
        for off in range(2):
            b = (_GNC - 2 + off) % _GR
            pltpu.make_async_copy(gab[b], ga_hbm.at[pl.ds(0, _GCH)], oasem[b]).wait()
            pltpu.make_async_copy(gcb[b], gc_hbm.at[pl.ds(0, _GCH)], ocsem[b]).wait()

    return k(a, c, src_p, dst_p)


def _sc_scatter(q, dstp, eb, bb):
    @functools.partial(
        pl.kernel,
        mesh=_mesh,
        out_type=jax.ShapeDtypeStruct((_NPAD * _D,), jnp.float32),
        scratch_types=[
            pltpu.VMEM((_NPT * _D,), jnp.float32),
            pltpu.VMEM((_SCH, _D), jnp.float32),
            pltpu.VMEM((_SCH, _D), jnp.float32),
            pltpu.VMEM((_SCH + 32,), jnp.int32),
            pltpu.VMEM((_SCH + 32,), jnp.int32),
            pltpu.VMEM((16,), jnp.int32),
            pltpu.VMEM((_D,), jnp.float32),
            pltpu.SemaphoreType.DMA,
            pltpu.SemaphoreType.DMA,
            pltpu.SemaphoreType.DMA,
            pltpu.SemaphoreType.DMA,
        ],
    )
    def k(q_hbm, dst_hbm, eb_hbm, bb_hbm, out_hbm,
          tbl, qb0, qb1, db0, db1, ebuf, bbuf,
          qsem0, qsem1, dsem0, dsem1):
        qb = (qb0, qb1)
        db = (db0, db1)
        qsem = (qsem0, qsem1)
        dsem = (dsem0, dsem1)
        wid = lax.axis_index("s") * _sc_info.num_cores + lax.axis_index("c")
        nlo = wid * _NPT
        pltpu.sync_copy(eb_hbm.at[pl.ds(pl.multiple_of(wid * 16, 8), 16)], ebuf)
        pltpu.sync_copy(bb_hbm, bbuf)
        ev = ebuf[pl.ds(0, _LANES)]
        elo = ev[0]
        ehi = ev[1]

        neg = jnp.full((_LANES,), -jnp.inf, jnp.float32)

        def initrow(r, _):
            tbl[pl.ds(r * _LANES, _LANES)] = neg
            return 0

        lax.fori_loop(0, _NPT * _VPR, initrow, 0)

        base8 = pl.multiple_of((elo // 8) * 8, 8)
        nchunks = (ehi - base8 + _SCH - 1) // _SCH

        def issue(ci, b):
            s = pl.multiple_of(base8 + ci * _SCH, 8)
            pltpu.async_copy(q_hbm.at[pl.ds(s, _SCH)], qb[b], qsem[b])
            pltpu.async_copy(
                dst_hbm.at[pl.ds(s, _SCH)], db[b].at[pl.ds(0, _SCH)], dsem[b])

        @pl.when(nchunks > 0)
        def _():
            issue(0, 0)

        @pl.when(nchunks > 1)
        def _():
            issue(1, 1)

        def outer(oo, _):
            for b in range(2):
                ci = 2 * oo + b

                @pl.when(ci < nchunks)
                def _():
                    pltpu.make_async_copy(
                        q_hbm.at[pl.ds(0, _SCH)], qb[b], qsem[b]).wait()
                    pltpu.make_async_copy(
                        dst_hbm.at[pl.ds(0, _SCH)],
                        db[b].at[pl.ds(0, _SCH)], dsem[b]).wait()
                    start = pl.multiple_of(base8 + ci * _SCH, 8)
                    i_lo = jnp.maximum(elo - start, 0)
                    i_hi = jnp.minimum(ehi - start, _SCH)

                    def flush(row, acc):
                        base = row * _D
                        for j in range(_VPR):
                            sl = pl.ds(base + j * _LANES, _LANES)
                            tbl[sl] = jnp.maximum(tbl[sl], acc[j])

                    zero = jnp.zeros((_LANES,), jnp.float32)

                    def group(gi, carry):
                        prev = carry[0]
                        acc = carry[1]
                        i0 = gi * _LANES
                        dvec = db[b][pl.ds(i0, _LANES)]
                        for lane in range(_LANES):
                            i = i0 + lane
                            d = dvec[lane]
                            valid = (i >= i_lo) & (i < i_hi)
                            is_new = valid & (d != prev)

                            @pl.when(is_new & (prev >= 0))
                            def _():
                                flush(prev - nlo, acc)

                            qrow = tuple(
                                qb[b][i, pl.ds(j * _LANES, _LANES)]
                                for j in range(_VPR))
                            acc = tuple(
                                jnp.where(
                                    is_new, qrow[j],
                                    jnp.where(valid,
                                              jnp.maximum(acc[j], qrow[j]),
                                              acc[j]))
                                for j in range(_VPR))
                            prev = jnp.where(is_new, d, prev)
                        return (prev, acc)

                    prev, acc = lax.fori_loop(
                        0, _SCH // _LANES, group,
                        (jnp.int32(-1), (zero,) * _VPR))

                    @pl.when(prev >= 0)
                    def _():
                        flush(prev - nlo, acc)

                    @pl.when(ci + 2 < nchunks)
                    def _():
                        issue(ci + 2, b)
            return 0

        lax.fori_loop(0, (nchunks + 1) // 2, outer, 0)

        def finrow(r, _):
            base = r * _D
            for j in range(_VPR):
                sl = pl.ds(base + j * _LANES, _LANES)
                bsl = pl.ds(j * _LANES, _LANES)
                tbl[sl] = jnp.maximum(tbl[sl] + bbuf[bsl], 0.0)
            return 0

        lax.fori_loop(0, _NPT, finrow, 0)
        pltpu.sync_copy(tbl, out_hbm.at[pl.ds(pl.multiple_of(nlo * _D, 8), _NPT * _D)])

    return k(q, dstp, eb, bb)


def kernel(x, pos, edge_index, W1a, b1a, W1b, b1b, W2a, b2a, W2b, b2b):
    src = edge_index[0]
    dst = edge_index[1]
    perm = jnp.argsort(dst, stable=False)
    dst_s = dst[perm].astype(jnp.int32)
    src_s = src[perm].astype(jnp.int32)
    bnd = jnp.searchsorted(
        dst_s, jnp.arange(_NW + 1, dtype=jnp.int32) * _NPT).astype(jnp.int32)
    eb = jnp.zeros((_NW, 16), jnp.int32)
    eb = eb.at[:, 0].set(bnd[:-1]).at[:, 1].set(bnd[1:]).reshape(-1)
    dstp = jnp.pad(dst_s, (0, _EPAD - _E))
    srcp = jnp.pad(src_s, (0, _EPAD - _E))
    posp = jnp.pad(pos, ((0, 0), (0, 125)))

    h = x
    for (Wa, ba, Wb, bb) in ((W1a, b1a, W1b, b1b), (W2a, b2a, W2b, b2b)):
        wah = Wa[:_D]
        wap = jnp.pad(Wa[_D:], ((0, 125), (0, 0)))
        a, cpart = _node_mm(h, posp, wah, wap, ba[None])
        ga, gc = _sc_gather(a, cpart, srcp, dstp)
        q = _edge_mm(ga, gc, Wb)
        h = _sc_scatter(q, dstp, eb, bb).reshape(_NPAD, _D)[:_N]
    return h
